# baseline (speedup 1.0000x reference)
import functools

import jax
import jax.numpy as jnp
from jax import lax
from jax.experimental import pallas as pl
from jax.experimental.pallas import tpu as pltpu
from jax.experimental.pallas import tpu_sc as plsc

ROWS, COLS = 16384, 312
OUT_COLS = 112
IN_SPAN = 256
NC, NS, L = 2, 16, 16
NW = NC * NS
ROWS_PER_W = ROWS // NW
R_BLK = 256
N_BLK = ROWS_PER_W // R_BLK
NT = OUT_COLS // L


def _sc_body(x_hbm, out_hbm, inbuf, outbuf):
    wid = lax.axis_index("s") * NC + lax.axis_index("c")
    base_w = wid * ROWS_PER_W
    col_even = lax.iota(jnp.int32, L) * 2

    for b in range(N_BLK):
        base = base_w + b * R_BLK
        pltpu.sync_copy(x_hbm.at[pl.ds(base, R_BLK), pl.ds(0, IN_SPAN)], inbuf)

        def row_fn(r, carry):
            rvec = jnp.full((L,), r, jnp.int32)
            for t in range(NT):
                v = plsc.load_gather(inbuf, [rvec, col_even + 2 * L * t])
                outbuf[r, pl.ds(L * t, L)] = v
            return carry

        lax.fori_loop(0, R_BLK, row_fn, 0)
        pltpu.sync_copy(outbuf, out_hbm.at[pl.ds(base, R_BLK)])


@jax.jit
def kernel(x):
    mesh = plsc.VectorSubcoreMesh(core_axis_name="c", subcore_axis_name="s")
    fn = pl.kernel(
        _sc_body,
        out_type=jax.ShapeDtypeStruct((ROWS, OUT_COLS), jnp.float32),
        mesh=mesh,
        scratch_types=[
            pltpu.VMEM((R_BLK, IN_SPAN), jnp.float32),
            pltpu.VMEM((R_BLK, OUT_COLS), jnp.float32),
        ],
        compiler_params=pltpu.CompilerParams(
            use_tc_tiling_on_sc=False, needs_layout_passes=False),
    )
    return fn(x)


# speedup vs baseline: 1.1252x; 1.1252x over previous
import functools

import jax
import jax.numpy as jnp
from jax import lax
from jax.experimental import pallas as pl
from jax.experimental.pallas import tpu as pltpu
from jax.experimental.pallas import tpu_sc as plsc

ROWS, COLS = 16384, 312
OUT_COLS = 112
IN_SPAN = 224
NC, NS, L = 2, 16, 16
NW = NC * NS
ROWS_PER_W = ROWS // NW
R_BLK = 256
N_BLK = ROWS_PER_W // R_BLK
NT = OUT_COLS // L


def _sc_body(x_hbm, out_hbm, inbuf, outbuf):
    wid = lax.axis_index("s") * NC + lax.axis_index("c")
    base_w = wid * ROWS_PER_W
    col_even = lax.iota(jnp.int32, L) * 2

    for b in range(N_BLK):
        base = base_w + b * R_BLK
        pltpu.sync_copy(x_hbm.at[pl.ds(base, R_BLK), pl.ds(0, IN_SPAN)], inbuf)

        @plsc.parallel_loop(0, R_BLK, unroll=8)
        def row_fn(r):
            row_in = inbuf.at[r]
            row_out = outbuf.at[r]
            for t in range(NT):
                v = plsc.load_gather(row_in, [col_even + 2 * L * t])
                row_out[pl.ds(L * t, L)] = v

        pltpu.sync_copy(outbuf, out_hbm.at[pl.ds(base, R_BLK)])


@jax.jit
def kernel(x):
    mesh = plsc.VectorSubcoreMesh(core_axis_name="c", subcore_axis_name="s")
    fn = pl.kernel(
        _sc_body,
        out_type=jax.ShapeDtypeStruct((ROWS, OUT_COLS), jnp.float32),
        mesh=mesh,
        scratch_types=[
            pltpu.VMEM((R_BLK, IN_SPAN), jnp.float32),
            pltpu.VMEM((R_BLK, OUT_COLS), jnp.float32),
        ],
        compiler_params=pltpu.CompilerParams(
            use_tc_tiling_on_sc=False, needs_layout_passes=False),
    )
    return fn(x)


# speedup vs baseline: 1.8508x; 1.6448x over previous
import functools

import jax
import jax.numpy as jnp
from jax import lax
from jax.experimental import pallas as pl
from jax.experimental.pallas import tpu as pltpu
from jax.experimental.pallas import tpu_sc as plsc

ROWS, COLS = 16384, 312
OUT_COLS = 112
IN_SPAN = 256
NC, NS, L = 2, 16, 16
NW = NC * NS
ROWS_PER_W = ROWS // NW
R_BLK = 256
N_BLK = ROWS_PER_W // R_BLK
NT = OUT_COLS // L


def _sc_body(x_hbm, out_hbm, inbuf, outbuf):
    wid = lax.axis_index("s") * NC + lax.axis_index("c")
    base_w = wid * ROWS_PER_W
    col_even = lax.iota(jnp.int32, L) * 2

    for b in range(N_BLK):
        base = base_w + b * R_BLK
        pltpu.sync_copy(x_hbm.at[pl.ds(base, R_BLK), pl.ds(0, IN_SPAN)], inbuf)

        @plsc.parallel_loop(0, R_BLK, unroll=8)
        def row_fn(r):
            rvec = jnp.full((L,), r, jnp.int32)
            for t in range(NT):
                v = plsc.load_gather(inbuf, [rvec, col_even + 2 * L * t])
                outbuf[r, pl.ds(L * t, L)] = v

        pltpu.sync_copy(outbuf, out_hbm.at[pl.ds(base, R_BLK)])


@jax.jit
def kernel(x):
    mesh = plsc.VectorSubcoreMesh(core_axis_name="c", subcore_axis_name="s")
    fn = pl.kernel(
        _sc_body,
        out_type=jax.ShapeDtypeStruct((ROWS, OUT_COLS), jnp.float32),
        mesh=mesh,
        scratch_types=[
            pltpu.VMEM((R_BLK, IN_SPAN), jnp.float32),
            pltpu.VMEM((R_BLK, OUT_COLS), jnp.float32),
        ],
        compiler_params=pltpu.CompilerParams(needs_layout_passes=False),
    )
    return fn(x)


# speedup vs baseline: 2.4558x; 1.3269x over previous
import jax
import jax.numpy as jnp
from jax import lax
from jax.experimental import pallas as pl
from jax.experimental.pallas import tpu as pltpu

ROWS, COLS = 16384, 312
OUT_COLS = 112
R_BLK = 2048


def _tc_body(x_ref, o_ref):
    r = lax.broadcasted_iota(jnp.int32, (COLS, OUT_COLS), 0)
    c = lax.broadcasted_iota(jnp.int32, (COLS, OUT_COLS), 1)
    sel = (r == 2 * c).astype(jnp.float32)
    o_ref[...] = jnp.dot(x_ref[...], sel, preferred_element_type=jnp.float32)


@jax.jit
def kernel(x):
    return pl.pallas_call(
        _tc_body,
        grid=(ROWS // R_BLK,),
        in_specs=[pl.BlockSpec((R_BLK, COLS), lambda i: (i, 0))],
        out_specs=pl.BlockSpec((R_BLK, OUT_COLS), lambda i: (i, 0)),
        out_shape=jax.ShapeDtypeStruct((ROWS, OUT_COLS), jnp.float32),
    )(x)
